# baseline (speedup 1.0000x reference)
import jax
import jax.numpy as jnp
from jax.experimental import pallas as pl
from jax.experimental.pallas import tpu as pltpu

_EPS = 1e-08
_BLOCK = 1024


def _route_block(hb, a_n):
    h_n = hb / jnp.maximum(
        jnp.sqrt(jnp.sum(hb * hb, axis=1, keepdims=True)), _EPS)
    scores = jax.lax.dot_general(
        h_n, a_n, (((1,), (1,)), ((), ())),
        preferred_element_type=jnp.float32)
    return jnp.argmax(scores, axis=1).astype(jnp.int32)


def _router_kernel(h_ref, a_ref, out_ref, buf, sems):
    nblk = h_ref.shape[0] // _BLOCK

    def copy(i, slot):
        return pltpu.make_async_copy(
            h_ref.at[pl.ds(i * _BLOCK, _BLOCK), :], buf.at[slot],
            sems.at[slot])

    a = a_ref[...]
    a_n = a / jnp.maximum(
        jnp.sqrt(jnp.sum(a * a, axis=1, keepdims=True)), _EPS)

    copy(0, 0).start()
    for i in range(nblk):
        slot = i % 2
        if i + 1 < nblk:
            copy(i + 1, 1 - slot).start()
        copy(i, slot).wait()
        out_ref[pl.ds(i * _BLOCK, _BLOCK), :] = (
            _route_block(buf[slot], a_n)[:, None])


def kernel(h, A):
    B, D = h.shape
    K = A.shape[0]
    return pl.pallas_call(
        _router_kernel,
        in_specs=[
            pl.BlockSpec(memory_space=pltpu.HBM),
            pl.BlockSpec((K, D), lambda: (0, 0)),
        ],
        out_specs=pl.BlockSpec((B, 1), lambda: (0, 0)),
        out_shape=jax.ShapeDtypeStruct((B, 1), jnp.int32),
        scratch_shapes=[
            pltpu.VMEM((2, _BLOCK, D), jnp.float32),
            pltpu.SemaphoreType.DMA((2,)),
        ],
        compiler_params=pltpu.CompilerParams(
            vmem_limit_bytes=100 * 1024 * 1024,
        ),
    )(h, A)
